# Initial kernel scaffold; baseline (speedup 1.0000x reference)
import jax
import jax.numpy as jnp
from jax.experimental import pallas as pl


def kernel(node_feats, edge_index, edge_feats, W_e, b_e, g_e, beta_e, W_m, b_m, g_m, beta_m, W_s, b_s, g_s, beta_s, g_n, beta_n):
    raise NotImplementedError("write your pallas kernel here")


# speedup vs baseline: 2.0295x; 2.0295x over previous
import functools

import jax
import jax.numpy as jnp
from jax import lax
from jax.experimental import pallas as pl
from jax.experimental.pallas import tpu as pltpu
from jax.experimental.pallas import tpu_sc as plsc

_NC = 2
_NS = 16
_L = 16
_NW = _NC * _NS
_EPS = 1e-5


def _ef_stats_body(ef_ref, we_ref, out_ref, s1, m2):
    i = pl.program_id(0)
    ef = ef_ref[...]

    @pl.when(i == 0)
    def _init():
        s1[...] = jnp.zeros_like(s1)
        m2[...] = jnp.zeros_like(m2)

    s1[...] += jnp.sum(ef, axis=0, keepdims=True)
    m2[...] += lax.dot_general(ef, ef, (((0,), (0,)), ((), ())),
                               preferred_element_type=jnp.float32)

    @pl.when(i == pl.num_programs(0) - 1)
    def _fin():
        e_total = pl.num_programs(0) * ef.shape[0]
        mean_ef = s1[...] / e_total
        cov = m2[...] / e_total - lax.dot_general(
            mean_ef, mean_ef, (((0,), (0,)), ((), ())),
            preferred_element_type=jnp.float32)
        w = we_ref[...]
        mean_z = jnp.dot(mean_ef, w, preferred_element_type=jnp.float32)
        cw = jnp.dot(cov, w, preferred_element_type=jnp.float32)
        var_z = jnp.sum(w * cw, axis=0, keepdims=True)
        out_ref[0:1, :] = mean_z
        out_ref[1:2, :] = var_z


def _ef_stats(edge_feats, W_e, tile):
    e, de = edge_feats.shape
    d = W_e.shape[1]
    grid = e // tile
    return pl.pallas_call(
        _ef_stats_body,
        grid=(grid,),
        in_specs=[
            pl.BlockSpec((tile, de), lambda i: (i, 0)),
            pl.BlockSpec((de, d), lambda i: (0, 0)),
        ],
        out_specs=pl.BlockSpec((2, d), lambda i: (0, 0)),
        out_shape=jax.ShapeDtypeStruct((2, d), jnp.float32),
        scratch_shapes=[
            pltpu.VMEM((1, de), jnp.float32),
            pltpu.VMEM((de, de), jnp.float32),
        ],
    )(edge_feats, W_e)


def _tables_body(nf_ref, ws_ref, wd_ref, t1_ref, t2_ref):
    nf = nf_ref[...]
    t1_ref[...] = jnp.dot(nf, ws_ref[...], preferred_element_type=jnp.float32)
    t2_ref[...] = jnp.dot(nf, wd_ref[...], preferred_element_type=jnp.float32)


def _tables(node_feats, w_src, w_dst, tile):
    n, d = node_feats.shape
    w2 = w_src.shape[1]
    grid = n // tile
    return pl.pallas_call(
        _tables_body,
        grid=(grid,),
        in_specs=[
            pl.BlockSpec((tile, d), lambda i: (i, 0)),
            pl.BlockSpec((d, w2), lambda i: (0, 0)),
            pl.BlockSpec((d, w2), lambda i: (0, 0)),
        ],
        out_specs=[
            pl.BlockSpec((tile, w2), lambda i: (i, 0)),
            pl.BlockSpec((tile, w2), lambda i: (i, 0)),
        ],
        out_shape=[
            jax.ShapeDtypeStruct((n, w2), jnp.float32),
            jax.ShapeDtypeStruct((n, w2), jnp.float32),
        ],
    )(node_feats, w_src, w_dst)


def _silu(x):
    return x * (1.0 / (1.0 + jnp.exp(-x)))


def _softplus(x):
    return jnp.maximum(x, 0.0) + jnp.log1p(jnp.exp(-jnp.abs(x)))


def _pre_stats_body(ef_ref, g_ref, wef_ref, bef_ref, w2_ref, b2_ref,
                    pre_ref, sums_ref):
    i = pl.program_id(0)
    hm = _silu(jnp.dot(ef_ref[...], wef_ref[...],
                       preferred_element_type=jnp.float32) + bef_ref[...])
    q = jnp.dot(hm, w2_ref[...], preferred_element_type=jnp.float32)
    pre = g_ref[...] + q + b2_ref[...]
    pre_ref[...] = pre

    @pl.when(i == 0)
    def _init():
        sums_ref[...] = jnp.zeros_like(sums_ref)

    sums_ref[0:1, :] += jnp.sum(pre, axis=0, keepdims=True)
    sums_ref[1:2, :] += jnp.sum(pre * pre, axis=0, keepdims=True)


def _pre_stats(edge_feats, g, wef, bef, w2, b2, tile):
    e, de = edge_feats.shape
    w_out = g.shape[1]
    d = wef.shape[1]
    grid = e // tile
    return pl.pallas_call(
        _pre_stats_body,
        grid=(grid,),
        in_specs=[
            pl.BlockSpec((tile, de), lambda i: (i, 0)),
            pl.BlockSpec((tile, w_out), lambda i: (i, 0)),
            pl.BlockSpec((de, d), lambda i: (0, 0)),
            pl.BlockSpec((1, d), lambda i: (0, 0)),
            pl.BlockSpec((d, w_out), lambda i: (0, 0)),
            pl.BlockSpec((1, w_out), lambda i: (0, 0)),
        ],
        out_specs=[
            pl.BlockSpec((tile, w_out), lambda i: (i, 0)),
            pl.BlockSpec((2, w_out), lambda i: (0, 0)),
        ],
        out_shape=[
            jax.ShapeDtypeStruct((e, w_out), jnp.float32),
            jax.ShapeDtypeStruct((2, w_out), jnp.float32),
        ],
    )(edge_feats, g, wef, bef, w2, b2)


def _act_body(pre_ref, sc_ref, sh_ref, u_ref):
    y = pre_ref[...] * sc_ref[...] + sh_ref[...]
    d = u_ref.shape[1]
    u_ref[...] = _silu(y[:, :d]) * _softplus(y[:, d:])


def _act(pre, sc, sh, tile):
    e, w_out = pre.shape
    d = w_out // 2
    grid = e // tile
    return pl.pallas_call(
        _act_body,
        grid=(grid,),
        in_specs=[
            pl.BlockSpec((tile, w_out), lambda i: (i, 0)),
            pl.BlockSpec((1, w_out), lambda i: (0, 0)),
            pl.BlockSpec((1, w_out), lambda i: (0, 0)),
        ],
        out_specs=pl.BlockSpec((tile, d), lambda i: (i, 0)),
        out_shape=jax.ShapeDtypeStruct((e, d), jnp.float32),
    )(pre, sc, sh)


def _final_stats_body(p0_ref, p1_ref, agg_ref, sums_ref):
    i = pl.program_id(0)
    agg = p0_ref[...] + p1_ref[...]
    agg_ref[...] = agg

    @pl.when(i == 0)
    def _init():
        sums_ref[...] = jnp.zeros_like(sums_ref)

    sums_ref[0:1, :] += jnp.sum(agg, axis=0, keepdims=True)
    sums_ref[1:2, :] += jnp.sum(agg * agg, axis=0, keepdims=True)


def _final_stats(p0, p1, n, tile):
    d = p0.shape[1]
    grid = n // tile
    return pl.pallas_call(
        _final_stats_body,
        grid=(grid,),
        in_specs=[
            pl.BlockSpec((tile, d), lambda i: (i, 0)),
            pl.BlockSpec((tile, d), lambda i: (i, 0)),
        ],
        out_specs=[
            pl.BlockSpec((tile, d), lambda i: (i, 0)),
            pl.BlockSpec((2, d), lambda i: (0, 0)),
        ],
        out_shape=[
            jax.ShapeDtypeStruct((n, d), jnp.float32),
            jax.ShapeDtypeStruct((2, d), jnp.float32),
        ],
    )(p0, p1)


def _final_out_body(agg_ref, nf_ref, sc_ref, sh_ref, out_ref):
    out_ref[...] = _softplus(agg_ref[...] * sc_ref[...] + sh_ref[...]
                             + nf_ref[...])


def _final_out(agg, node_feats, sc, sh, tile):
    n, d = agg.shape
    grid = n // tile
    return pl.pallas_call(
        _final_out_body,
        grid=(grid,),
        in_specs=[
            pl.BlockSpec((tile, d), lambda i: (i, 0)),
            pl.BlockSpec((tile, d), lambda i: (i, 0)),
            pl.BlockSpec((1, d), lambda i: (0, 0)),
            pl.BlockSpec((1, d), lambda i: (0, 0)),
        ],
        out_specs=pl.BlockSpec((tile, d), lambda i: (i, 0)),
        out_shape=jax.ShapeDtypeStruct((n, d), jnp.float32),
    )(agg, node_feats, sc, sh)


def _gather_add_sc(t1, t2, src, dst):
    n, w = t1.shape
    e = src.shape[0]
    epw = e // _NW
    ch = 80
    nch = epw // ch
    mesh = plsc.VectorSubcoreMesh(core_axis_name="c", subcore_axis_name="s")

    @functools.partial(
        pl.kernel,
        out_type=jax.ShapeDtypeStruct((e, w), jnp.float32),
        mesh=mesh,
        scratch_types=[
            pltpu.VMEM((ch,), jnp.int32),
            pltpu.VMEM((ch,), jnp.int32),
            pltpu.VMEM((ch, w), jnp.float32),
            pltpu.VMEM((ch, w), jnp.float32),
            pltpu.SemaphoreType.DMA,
            pltpu.SemaphoreType.DMA,
        ],
    )
    def gk(t1_h, t2_h, src_h, dst_h, out_h, sidx, didx, abuf, bbuf, sem1, sem2):
        cid = lax.axis_index("c")
        sid = lax.axis_index("s")
        base = (sid * _NC + cid) * epw

        def chunk(i, carry):
            off = base + i * ch
            pltpu.sync_copy(src_h.at[pl.ds(off, ch)], sidx)
            pltpu.sync_copy(dst_h.at[pl.ds(off, ch)], didx)
            c1 = pltpu.async_copy(t1_h.at[sidx], abuf, sem1)
            c2 = pltpu.async_copy(t2_h.at[didx], bbuf, sem2)
            c1.wait()
            c2.wait()

            def row(r, c_):
                for cix in range(w // _L):
                    sl = pl.ds(cix * _L, _L)
                    abuf[r, sl] = abuf[r, sl] + bbuf[r, sl]
                return c_

            lax.fori_loop(0, ch, row, 0)
            pltpu.sync_copy(abuf, out_h.at[pl.ds(off, ch)])
            return carry

        lax.fori_loop(0, nch, chunk, 0)

    return gk(t1, t2, src, dst)


def _scatter_sc(u, dst, zeros):
    e, d = u.shape
    n = zeros.shape[0]
    epc = e // _NC
    ept = epc // _NS
    ch = 80
    nch = ept // ch
    rpt = n // _NS
    mesh = plsc.VectorSubcoreMesh(core_axis_name="c", subcore_axis_name="s")

    @functools.partial(
        pl.kernel,
        out_type=jax.ShapeDtypeStruct((_NC * n, d), jnp.float32),
        mesh=mesh,
        scratch_types=[
            pltpu.VMEM((ch,), jnp.int32),
            pltpu.VMEM((ch, d), jnp.float32),
            pltpu.VMEM_SHARED((n, d), jnp.float32),
        ],
    )
    def sk(u_h, dst_h, z_h, out_h, didx, ubuf, acc):
        cid = lax.axis_index("c")
        sid = lax.axis_index("s")
        r0 = sid * rpt
        pltpu.sync_copy(z_h.at[pl.ds(r0, rpt)], acc.at[pl.ds(r0, rpt)])
        plsc.subcore_barrier()
        base = cid * epc + sid * ept

        def chunk(i, carry):
            off = base + i * ch
            pltpu.sync_copy(dst_h.at[pl.ds(off, ch)], didx)
            pltpu.sync_copy(u_h.at[pl.ds(off, ch)], ubuf)
            pltpu.sync_copy(ubuf, acc.at[didx], add=True)
            return carry

        lax.fori_loop(0, nch, chunk, 0)
        plsc.subcore_barrier()
        pltpu.sync_copy(acc.at[pl.ds(r0, rpt)],
                        out_h.at[pl.ds(cid * n + r0, rpt)])

    return sk(u, dst, zeros)


def kernel(node_feats, edge_index, edge_feats, W_e, b_e, g_e, beta_e,
           W_m, b_m, g_m, beta_m, W_s, b_s, g_s, beta_s, g_n, beta_n):
    n, d = node_feats.shape
    e = edge_index.shape[1]
    src = edge_index[0]
    dst = edge_index[1]

    te = 2560
    tn = 2000

    stats_z = _ef_stats(edge_feats, W_e, te)
    s_e = g_e * lax.rsqrt(stats_z[1] + _EPS)
    wef = W_e * s_e[None, :]
    bef = (beta_e - stats_z[0] * s_e)[None, :]

    w_src = jnp.concatenate([W_m[:d], W_s[:d]], axis=1)
    w_dst = jnp.concatenate([W_m[d:2 * d], W_s[d:2 * d]], axis=1)
    w2 = jnp.concatenate([W_m[2 * d:], W_s[2 * d:]], axis=1)
    b2 = jnp.concatenate([b_m, b_s])[None, :]

    t1, t2 = _tables(node_feats, w_src, w_dst, tn)
    g = _gather_add_sc(t1, t2, src, dst)
    pre, sums = _pre_stats(edge_feats, g, wef, bef, w2, b2, te)

    mean = sums[0] / e
    var = sums[1] / e - mean * mean
    sc = jnp.concatenate([g_m, g_s]) * lax.rsqrt(var + _EPS)
    sh = jnp.concatenate([beta_m, beta_s]) - mean * sc

    u = _act(pre, sc[None, :], sh[None, :], te)
    n_pad = ((n + 8 * _NS - 1) // (8 * _NS)) * (8 * _NS)
    partials = _scatter_sc(u, dst, jnp.zeros((n_pad, d), jnp.float32))
    p0 = partials[:n]
    p1 = partials[n_pad:n_pad + n]

    agg, nsums = _final_stats(p0, p1, n, tn)
    meann = nsums[0] / n
    varn = nsums[1] / n - meann * meann
    scn = g_n * lax.rsqrt(varn + _EPS)
    shn = beta_n - meann * scn
    return _final_out(agg, node_feats, scn[None, :], shn[None, :], tn)
